# baseline (speedup 1.0000x reference)
import functools

import jax
import jax.numpy as jnp
from jax import lax
from jax.experimental import pallas as pl
from jax.experimental.pallas import tpu as pltpu
from jax.experimental.pallas import tpu_sc as plsc

B, W, L = 1024, 64, 3
N = B * W * L
DIM = 64
NW = 32
PER_W = N // NW
CHUNK = 128
NCH = PER_W // CHUNK

TWO_PI = 6.283185307179586
INV_TWO_PI = 0.15915494309189535
PI = 3.141592653589793
C0 = -0.9999992109801177
C1 = 0.499994213707783
C2 = -0.04165977794574207
C3 = 0.001385879013978696
C4 = -2.420294256311692e-05
C5 = 2.197296441102012e-07

_MESH = plsc.VectorSubcoreMesh(core_axis_name="c", subcore_axis_name="s")


@functools.partial(
    pl.kernel,
    out_type=jax.ShapeDtypeStruct((N, 3 * DIM), jnp.float32),
    mesh=_MESH,
    compiler_params=pltpu.CompilerParams(use_tc_tiling_on_sc=False,
                                        needs_layout_passes=False),
    scratch_types=[
        pltpu.VMEM((NCH, CHUNK), jnp.int32),
        pltpu.VMEM((NCH, CHUNK), jnp.int32),
        pltpu.VMEM((NCH, CHUNK), jnp.float32),
        pltpu.VMEM((4, 16), jnp.float32),
        pltpu.VMEM((4, 16), jnp.float32),
        pltpu.VMEM((CHUNK, DIM), jnp.float32),
        pltpu.VMEM((CHUNK, DIM), jnp.float32),
        pltpu.VMEM((CHUNK, DIM), jnp.float32),
        pltpu.SemaphoreType.DMA,
        pltpu.SemaphoreType.DMA,
    ],
)
def _cawn_sc(nr_hbm, er_hbm, tr_hbm, ntab_hbm, etab_hbm, fr_hbm, ph_hbm,
             out_hbm, nidx, eidx, tbuf, frv, phv, nrows, erows, trows,
             nsem, esem):
    wid = lax.axis_index("s") * 2 + lax.axis_index("c")
    pltpu.sync_copy(nr_hbm.at[wid], nidx)
    pltpu.sync_copy(er_hbm.at[wid], eidx)
    pltpu.sync_copy(tr_hbm.at[wid], tbuf)
    pltpu.sync_copy(fr_hbm, frv)
    pltpu.sync_copy(ph_hbm, phv)

    def chunk_body(c, carry):
        ncp = pltpu.async_copy(ntab_hbm.at[nidx.at[c]], nrows, nsem)
        ecp = pltpu.async_copy(etab_hbm.at[eidx.at[c]], erows, esem)

        def step(i, carry2):
            cvec = jnp.full((16,), c, jnp.int32)
            ivec = jnp.full((16,), i, jnp.int32)
            tv = plsc.load_gather(tbuf, [cvec, ivec])
            for kk in range(4):
                w = frv[kk, :]
                p = phv[kk, :]
                x = tv * w + p
                n = (x * INV_TWO_PI).astype(jnp.int32)
                s = x - n.astype(jnp.float32) * TWO_PI - PI
                u = s * s
                y = ((((C5 * u + C4) * u + C3) * u + C2) * u + C1) * u + C0
                trows[i, pl.ds(16 * kk, 16)] = y
            return carry2

        lax.fori_loop(0, CHUNK, step, 0)
        ncp.wait()
        ecp.wait()
        gbase = wid * PER_W + c * CHUNK
        pltpu.sync_copy(nrows, out_hbm.at[pl.ds(gbase, CHUNK), pl.ds(0, DIM)])
        pltpu.sync_copy(erows, out_hbm.at[pl.ds(gbase, CHUNK), pl.ds(DIM, DIM)])
        pltpu.sync_copy(trows, out_hbm.at[pl.ds(gbase, CHUNK), pl.ds(2 * DIM, DIM)])
        return carry

    lax.fori_loop(0, NCH, chunk_body, 0)


def kernel(node_records, edge_records, t_records, node_table, edge_table,
           basis_freq, phase):
    nr = node_records.reshape(NW, NCH, CHUNK).astype(jnp.int32)
    er = edge_records.reshape(NW, NCH, CHUNK).astype(jnp.int32)
    tr = t_records.reshape(NW, NCH, CHUNK)
    fr = basis_freq.reshape(4, 16)
    ph = phase.reshape(4, 16)
    out = _cawn_sc(nr, er, tr, node_table, edge_table, fr, ph)
    return out.reshape(B, W, L, 3 * DIM)


# speedup vs baseline: 1.7084x; 1.7084x over previous
import functools

import jax
import jax.numpy as jnp
from jax import lax
from jax.experimental import pallas as pl
from jax.experimental.pallas import tpu as pltpu
from jax.experimental.pallas import tpu_sc as plsc

B, W, L = 1024, 64, 3
N = B * W * L
DIM = 64
NW = 32
PER_W = N // NW
CHUNK = 128
NCH = PER_W // CHUNK

TWO_PI = 6.283185307179586
PI = 3.141592653589793
INV_TWO_PI = 0.15915494309189535
C0 = -0.9999992109801177
C1 = 0.499994213707783
C2 = -0.04165977794574207
C3 = 0.001385879013978696
C4 = -2.420294256311692e-05
C5 = 2.197296441102012e-07

_MESH = plsc.VectorSubcoreMesh(core_axis_name="c", subcore_axis_name="s")


@functools.partial(
    pl.kernel,
    out_type=jax.ShapeDtypeStruct((N, 3 * DIM), jnp.float32),
    mesh=_MESH,
    compiler_params=pltpu.CompilerParams(use_tc_tiling_on_sc=False,
                                        needs_layout_passes=False),
    scratch_types=[
        pltpu.VMEM((NCH, CHUNK), jnp.int32),
        pltpu.VMEM((NCH, CHUNK), jnp.int32),
        pltpu.VMEM((NCH, CHUNK), jnp.float32),
        pltpu.VMEM((4, 16), jnp.float32),
        pltpu.VMEM((4, 16), jnp.float32),
        pltpu.VMEM((4, 16), jnp.float32),
        pltpu.VMEM((4, 16), jnp.float32),
        [pltpu.VMEM((CHUNK, DIM), jnp.float32) for _ in range(2)],
        [pltpu.VMEM((CHUNK, DIM), jnp.float32) for _ in range(2)],
        [pltpu.VMEM((CHUNK, DIM), jnp.float32) for _ in range(2)],
        [pltpu.SemaphoreType.DMA for _ in range(2)],
        [pltpu.SemaphoreType.DMA for _ in range(2)],
        [pltpu.SemaphoreType.DMA for _ in range(2)],
    ],
)
def _cawn_sc(nr_hbm, er_hbm, tr_hbm, ntab_hbm, etab_hbm, fr_hbm, ph_hbm,
             w2_hbm, b2_hbm, out_hbm, nidx, eidx, tbuf, frv, phv, w2v, b2v,
             nrows, erows, trows, nsem, esem, osem):
    wid = lax.axis_index("s") * 2 + lax.axis_index("c")
    pltpu.sync_copy(nr_hbm.at[wid], nidx)
    pltpu.sync_copy(er_hbm.at[wid], eidx)
    pltpu.sync_copy(tr_hbm.at[wid], tbuf)
    pltpu.sync_copy(fr_hbm, frv)
    pltpu.sync_copy(ph_hbm, phv)
    pltpu.sync_copy(w2_hbm, w2v)
    pltpu.sync_copy(b2_hbm, b2v)

    def issue_gathers(c, b):
        pltpu.async_copy(ntab_hbm.at[nidx.at[c]], nrows[b], nsem[b])
        pltpu.async_copy(etab_hbm.at[eidx.at[c]], erows[b], esem[b])

    def wait_gathers(c, b):
        pltpu.make_async_copy(ntab_hbm.at[nidx.at[c]], nrows[b], nsem[b]).wait()
        pltpu.make_async_copy(etab_hbm.at[eidx.at[c]], erows[b], esem[b]).wait()

    def issue_out(c, b):
        gb = wid * PER_W + c * CHUNK
        pltpu.async_copy(nrows[b], out_hbm.at[pl.ds(gb, CHUNK), pl.ds(0, DIM)], osem[b])
        pltpu.async_copy(erows[b], out_hbm.at[pl.ds(gb, CHUNK), pl.ds(DIM, DIM)], osem[b])
        pltpu.async_copy(trows[b], out_hbm.at[pl.ds(gb, CHUNK), pl.ds(2 * DIM, DIM)], osem[b])

    def drain_out(c, b):
        gb = wid * PER_W + c * CHUNK
        pltpu.make_async_copy(nrows[b], out_hbm.at[pl.ds(gb, CHUNK), pl.ds(0, DIM)], osem[b]).wait()
        pltpu.make_async_copy(erows[b], out_hbm.at[pl.ds(gb, CHUNK), pl.ds(DIM, DIM)], osem[b]).wait()
        pltpu.make_async_copy(trows[b], out_hbm.at[pl.ds(gb, CHUNK), pl.ds(2 * DIM, DIM)], osem[b]).wait()

    def compute(c, b):
        tdst = trows[b]
        cvec = jnp.full((16,), c, jnp.int32)

        @plsc.parallel_loop(0, CHUNK, unroll=8)
        def _step(i):
            tv = plsc.load_gather(tbuf, [cvec, jnp.full((16,), i, jnp.int32)])
            for kk in range(4):
                x = tv * frv[kk, :] + phv[kk, :]
                q = tv * w2v[kk, :] + b2v[kk, :]
                s = x - q.astype(jnp.int32).astype(jnp.float32) * TWO_PI
                u = s * s
                y = ((((C5 * u + C4) * u + C3) * u + C2) * u + C1) * u + C0
                tdst[i, pl.ds(16 * kk, 16)] = y

    def phase(c, b, first=False, last=False):
        compute(c, b)
        if not first:
            drain_out(c - 1, 1 - b)
        if not last:
            issue_gathers(c + 1, 1 - b)
        wait_gathers(c, b)
        issue_out(c, b)

    issue_gathers(0, 0)
    phase(0, 0, first=True)
    phase(1, 1)

    def pair(cc, carry):
        phase(2 * cc, 0)
        phase(2 * cc + 1, 1)
        return carry

    lax.fori_loop(1, NCH // 2 - 1, pair, 0)
    phase(NCH - 2, 0)
    phase(NCH - 1, 1, last=True)
    drain_out(NCH - 1, 1)


def kernel(node_records, edge_records, t_records, node_table, edge_table,
           basis_freq, phase):
    nr = node_records.reshape(NW, NCH, CHUNK).astype(jnp.int32)
    er = edge_records.reshape(NW, NCH, CHUNK).astype(jnp.int32)
    tr = t_records.reshape(NW, NCH, CHUNK)
    fr = basis_freq.reshape(4, 16)
    ph = (phase - PI).reshape(4, 16)
    w2 = (basis_freq * INV_TWO_PI).reshape(4, 16)
    b2 = (phase * INV_TWO_PI).reshape(4, 16)
    out = _cawn_sc(nr, er, tr, node_table, edge_table, fr, ph, w2, b2)
    return out.reshape(B, W, L, 3 * DIM)


# speedup vs baseline: 2.4358x; 1.4258x over previous
import functools

import jax
import jax.numpy as jnp
from jax import lax
from jax.experimental import pallas as pl
from jax.experimental.pallas import tpu as pltpu
from jax.experimental.pallas import tpu_sc as plsc

B, W, L = 1024, 64, 3
DIM = 64
NW = 32
NPAIR = W * L // NW
NBT = B // 128
NCH = NPAIR * NBT

TWO_PI = 6.283185307179586
PI = 3.141592653589793
INV_TWO_PI = 0.15915494309189535
C0 = -0.9999992109801177
C1 = 0.499994213707783
C2 = -0.04165977794574207
C3 = 0.001385879013978696
C4 = -2.420294256311692e-05
C5 = 2.197296441102012e-07

_MESH = plsc.VectorSubcoreMesh(core_axis_name="c", subcore_axis_name="s")


@functools.partial(
    pl.kernel,
    out_type=jax.ShapeDtypeStruct((W, L, 3 * DIM // 8, B // 128, 8, 128),
                                  jnp.float32),
    mesh=_MESH,
    compiler_params=pltpu.CompilerParams(use_tc_tiling_on_sc=False,
                                        needs_layout_passes=False),
    scratch_types=[
        pltpu.VMEM((NPAIR, NBT, 128), jnp.int32),
        pltpu.VMEM((NPAIR, NBT, 128), jnp.int32),
        pltpu.VMEM((NPAIR, NBT, 128), jnp.float32),
        pltpu.VMEM((DIM,), jnp.float32),
        pltpu.VMEM((DIM,), jnp.float32),
        pltpu.VMEM((DIM,), jnp.float32),
        pltpu.VMEM((DIM,), jnp.float32),
        [pltpu.VMEM((128, DIM), jnp.float32) for _ in range(2)],
        [pltpu.VMEM((128, DIM), jnp.float32) for _ in range(2)],
        [pltpu.VMEM((8, 8, 128), jnp.float32) for _ in range(2)],
        [pltpu.VMEM((8, 8, 128), jnp.float32) for _ in range(2)],
        [pltpu.VMEM((8, 8, 128), jnp.float32) for _ in range(2)],
        [pltpu.SemaphoreType.DMA for _ in range(2)],
        [pltpu.SemaphoreType.DMA for _ in range(2)],
        [pltpu.SemaphoreType.DMA for _ in range(2)],
    ],
)
def _cawn_sc(nr_hbm, er_hbm, tr_hbm, ntab_hbm, etab_hbm, fr_hbm, ph_hbm,
             w2_hbm, b2_hbm, out_hbm, nidx, eidx, tbuf,
             frs, phs, w2s, b2s, nrows, erows, ntile, etile, ttile,
             nsem, esem, osem):
    wid = lax.axis_index("s") * 2 + lax.axis_index("c")

    pltpu.sync_copy(fr_hbm, frs)
    pltpu.sync_copy(ph_hbm, phs)
    pltpu.sync_copy(w2_hbm, w2s)
    pltpu.sync_copy(b2_hbm, b2s)

    for pq in range(NPAIR):
        q = wid * NPAIR + pq
        ll, ww = q // W, q % W
        wt, wi = ww // 8, ww % 8
        pltpu.sync_copy(nr_hbm.at[ll, wt, :, wi, :], nidx.at[pq])
        pltpu.sync_copy(er_hbm.at[ll, wt, :, wi, :], eidx.at[pq])
        pltpu.sync_copy(tr_hbm.at[ll, wt, :, wi, :], tbuf.at[pq])

    iota = lax.iota(jnp.int32, 16)

    def issue_gathers(c, b):
        pq, bt = c // NBT, c % NBT
        pltpu.async_copy(ntab_hbm.at[nidx.at[pq, bt]], nrows[b], nsem[b])
        pltpu.async_copy(etab_hbm.at[eidx.at[pq, bt]], erows[b], esem[b])

    def wait_gathers(c, b):
        pq, bt = c // NBT, c % NBT
        pltpu.make_async_copy(ntab_hbm.at[nidx.at[pq, bt]], nrows[b], nsem[b]).wait()
        pltpu.make_async_copy(etab_hbm.at[eidx.at[pq, bt]], erows[b], esem[b]).wait()

    def _out_slices(c):
        pq, bt = c // NBT, c % NBT
        q = wid * NPAIR + pq
        ll, ww = q // W, q % W
        return (out_hbm.at[ww, ll, pl.ds(0, 8), bt],
                out_hbm.at[ww, ll, pl.ds(8, 8), bt],
                out_hbm.at[ww, ll, pl.ds(16, 8), bt])

    def issue_out(c, b):
        sn, se, st = _out_slices(c)
        pltpu.async_copy(ntile[b], sn, osem[b])
        pltpu.async_copy(etile[b], se, osem[b])
        pltpu.async_copy(ttile[b], st, osem[b])

    def drain_out(c, b):
        sn, se, st = _out_slices(c)
        pltpu.make_async_copy(ntile[b], sn, osem[b]).wait()
        pltpu.make_async_copy(etile[b], se, osem[b]).wait()
        pltpu.make_async_copy(ttile[b], st, osem[b]).wait()

    def compute_time(c, b):
        pq, bt = c // NBT, c % NBT
        tdst = ttile[b]

        @plsc.parallel_loop(0, DIM, unroll=2)
        def _f(f):
            fv = jnp.full((16,), f, jnp.int32)
            w = plsc.load_gather(frs, [fv])
            p = plsc.load_gather(phs, [fv])
            w2 = plsc.load_gather(w2s, [fv])
            b2 = plsc.load_gather(b2s, [fv])
            ft, fi = f // 8, f % 8
            for j in range(8):
                tv = tbuf[pq, bt, pl.ds(16 * j, 16)]
                x = tv * w + p
                qq = tv * w2 + b2
                s = x - qq.astype(jnp.int32).astype(jnp.float32) * TWO_PI
                u = s * s
                y = ((((C5 * u + C4) * u + C3) * u + C2) * u + C1) * u + C0
                tdst[ft, fi, pl.ds(16 * j, 16)] = y

    def transpose_tiles(b):
        src_dst = ((nrows[b], ntile[b]), (erows[b], etile[b]))

        @plsc.parallel_loop(0, DIM, unroll=2)
        def _f(f):
            ft, fi = f // 8, f % 8
            fv = jnp.full((16,), f, jnp.int32)
            for src, dst in src_dst:
                for j in range(8):
                    v = plsc.load_gather(src, [iota + 16 * j, fv])
                    dst[ft, fi, pl.ds(16 * j, 16)] = v

    def phase_step(c, b, first=False, last=False):
        if not last:
            issue_gathers(c + 1, 1 - b)
        compute_time(c, b)
        wait_gathers(c, b)
        transpose_tiles(b)
        if not first:
            drain_out(c - 1, 1 - b)
        issue_out(c, b)

    issue_gathers(0, 0)
    phase_step(0, 0, first=True)
    phase_step(1, 1)

    def pair_steps(cc, carry):
        phase_step(2 * cc, 0)
        phase_step(2 * cc + 1, 1)
        return carry

    lax.fori_loop(1, NCH // 2 - 1, pair_steps, 0)
    phase_step(NCH - 2, 0)
    phase_step(NCH - 1, 1, last=True)
    drain_out(NCH - 1, 1)


def _records_view(x):
    return (x.transpose(2, 1, 0)
            .reshape(L, W // 8, 8, B // 128, 128)
            .transpose(0, 1, 3, 2, 4))


def kernel(node_records, edge_records, t_records, node_table, edge_table,
           basis_freq, phase):
    nr = _records_view(node_records.astype(jnp.int32))
    er = _records_view(edge_records.astype(jnp.int32))
    tr = _records_view(t_records)
    fr = basis_freq
    ph = phase - PI
    w2 = basis_freq * INV_TWO_PI
    b2 = phase * INV_TWO_PI
    out6 = _cawn_sc(nr, er, tr, node_table, edge_table, fr, ph, w2, b2)
    return out6.transpose(3, 5, 0, 1, 2, 4).reshape(B, W, L, 3 * DIM)
